# baseline (speedup 1.0000x reference)
import functools

import jax
import jax.numpy as jnp
from jax import lax
from jax.experimental import pallas as pl
from jax.experimental.pallas import tpu as pltpu
from jax.experimental.pallas import tpu_sc as plsc

K = 16


def _stem_body(x_ref, w0_ref, b0_ref, w1_ref, b1_ref, w2_ref, b2_ref, o_ref):
    x = x_ref[...]
    h = jax.nn.relu(jnp.dot(x, w0_ref[...], preferred_element_type=jnp.float32) + b0_ref[...])
    h = jax.nn.relu(jnp.dot(h, w1_ref[...], preferred_element_type=jnp.float32) + b1_ref[...])
    h = jax.nn.relu(jnp.dot(h, w2_ref[...], preferred_element_type=jnp.float32) + b2_ref[...])
    o_ref[...] = h


def _stem(xyz0, pw):
    B, N, _ = xyz0.shape
    (w0, b0), (w1, b1), (w2, b2) = pw
    x = xyz0.reshape(B * N, 3)
    out = pl.pallas_call(
        _stem_body,
        out_shape=jax.ShapeDtypeStruct((B * N, 16), jnp.float32),
    )(x, w0, b0, w1, b1, w2, b2)
    return out.reshape(B, N, 16)


def _ce(v, ix, i, j):
    p = v[i] > v[j]
    vi = jnp.where(p, v[j], v[i])
    vj = jnp.where(p, v[i], v[j])
    xi = jnp.where(p, ix[j], ix[i])
    xj = jnp.where(p, ix[i], ix[j])
    v[i], v[j], ix[i], ix[j] = vi, vj, xi, xj


def _bitonic_sort16(v, ix):
    for k in (2, 4, 8, 16):
        step = k // 2
        while step >= 1:
            for i in range(16):
                l = i ^ step
                if l > i:
                    if (i & k) == 0:
                        _ce(v, ix, i, l)
                    else:
                        _ce(v, ix, l, i)
            step //= 2


def _bitonic_merge16(v, ix):
    for step in (8, 4, 2, 1):
        for i in range(16):
            l = i ^ step
            if l > i:
                _ce(v, ix, i, l)


def _knn_body(qT_ref, r_ref, o_ref, *, N, bs, n_off_stride):
    b = pl.program_id(0)
    qT = qT_ref[0]
    r = r_ref[0]
    rr = jnp.sum(r * r, axis=1, keepdims=True)
    qq = jnp.sum(qT * qT, axis=0, keepdims=True)
    dT = rr - 2.0 * jnp.dot(r, qT, preferred_element_type=jnp.float32) + qq

    G = N // 16
    base = b * n_off_stride
    giota = lax.broadcasted_iota(jnp.int32, (G, bs), 0)
    v = [dT[j * G:(j + 1) * G, :] for j in range(16)]
    ix = [giota + (j * G + base) for j in range(16)]

    _bitonic_sort16(v, ix)

    g = G
    while g > 1:
        h = g // 2
        a_v = [v[j][:h, :] for j in range(16)]
        a_x = [ix[j][:h, :] for j in range(16)]
        b_v = [v[j][h:, :] for j in range(16)]
        b_x = [ix[j][h:, :] for j in range(16)]
        nv, nx = [], []
        for j in range(16):
            bv, bx = b_v[15 - j], b_x[15 - j]
            p = a_v[j] > bv
            nv.append(jnp.where(p, bv, a_v[j]))
            nx.append(jnp.where(p, bx, a_x[j]))
        v, ix = nv, nx
        _bitonic_merge16(v, ix)
        g = h

    o_ref[...] = jnp.concatenate(ix, axis=0).astype(jnp.int32)


def _knn(q, r, flat_offset_stride, bs=256):
    B, S, _ = q.shape
    N = r.shape[1]
    qT = q.transpose(0, 2, 1)
    body = functools.partial(_knn_body, N=N, bs=bs, n_off_stride=flat_offset_stride)
    out = pl.pallas_call(
        body,
        grid=(B, S // bs),
        in_specs=[
            pl.BlockSpec((1, 3, bs), lambda b, s: (b, 0, s)),
            pl.BlockSpec((1, N, 3), lambda b, s: (b, 0, 0)),
        ],
        out_specs=pl.BlockSpec((K, bs), lambda b, s, _S=S, _bs=bs: (0, b * (_S // _bs) + s)),
        out_shape=jax.ShapeDtypeStruct((K, B * S), jnp.int32),
    )(qT, r)
    return out


def _sc_gather(table, idx):
    R, D = table.shape
    M = idx.shape[0]
    NW = 32
    assert M % NW == 0
    b_per_w = M // NW
    chunk = b_per_w
    while chunk * D * 4 > 256 * 1024:
        chunk //= 2
    assert b_per_w % chunk == 0 and chunk % 8 == 0
    n_ch = b_per_w // chunk
    mesh = plsc.VectorSubcoreMesh(core_axis_name="c", subcore_axis_name="s")

    def body(table_hbm, idx_hbm, out_hbm, idx_v, rows_v, sem):
        wid = lax.axis_index("s") * 2 + lax.axis_index("c")
        base = wid * b_per_w
        pltpu.sync_copy(idx_hbm.at[pl.ds(base, b_per_w)], idx_v)
        for ch in range(n_ch):
            pltpu.async_copy(
                table_hbm.at[idx_v.at[pl.ds(ch * chunk, chunk)]], rows_v, sem
            ).wait()
            pltpu.sync_copy(rows_v, out_hbm.at[pl.ds(base + ch * chunk, chunk)])

    f = pl.kernel(
        body,
        out_type=jax.ShapeDtypeStruct((M, D), jnp.float32),
        mesh=mesh,
        compiler_params=pltpu.CompilerParams(use_tc_tiling_on_sc=False),
        scratch_types=[
            pltpu.VMEM((b_per_w,), jnp.int32),
            pltpu.VMEM((chunk, D), jnp.float32),
            pltpu.SemaphoreType.DMA,
        ],
    )
    return f(table, idx)


def _pad_cols(x, D):
    pad = D - x.shape[-1]
    if pad:
        x = jnp.concatenate([x, jnp.zeros(x.shape[:-1] + (pad,), x.dtype)], -1)
    return x


def _round16(n):
    return (n + 15) // 16 * 16


def _wnet(gx, w1, b1, w2, b2):
    w = jax.nn.relu(jnp.dot(gx, w1, preferred_element_type=jnp.float32) + b1)
    return jax.nn.relu(jnp.dot(w, w2, preferred_element_type=jnp.float32) + b2)


def _agg(rows, nx, w1, b1, w2, b2, C, bs):
    gx = (rows[:, :, C:C + 3] - nx[None, :, :]).reshape(K * bs, 3)
    w = _wnet(gx, w1, b1, w2, b2).reshape(K, bs, 4)
    gf = rows[:, :, :C]
    gf4 = jnp.concatenate([gf] * 4, axis=2)
    w4 = jnp.concatenate(
        [jnp.broadcast_to(w[:, :, m:m + 1], (K, bs, C)) for m in range(4)],
        axis=2)
    return jnp.sum(gf4 * w4, axis=0)


def _conv_interp_body(rows_ref, nx_ref, w1_ref, b1_ref, w2_ref, b2_ref,
                      wl_ref, bl_ref, wd_ref, bd_ref, f1_ref, hd_ref, *, C, bs):
    h = _agg(rows_ref[...], nx_ref[...], w1_ref[...], b1_ref[...],
             w2_ref[...], b2_ref[...], C, bs)
    f1 = jax.nn.relu(jnp.dot(h, wl_ref[...], preferred_element_type=jnp.float32) + bl_ref[...])
    f1_ref[...] = f1
    hd_ref[...] = jax.nn.relu(jnp.dot(f1, wd_ref[...], preferred_element_type=jnp.float32) + bd_ref[...])


def _conv_pc_body(rows_ref, nx_ref, f1_ref, w1_ref, b1_ref, w2_ref, b2_ref,
                  wl_ref, bl_ref, wu_ref, bu_ref, o_ref, *, C, bs):
    h = _agg(rows_ref[...], nx_ref[...], w1_ref[...], b1_ref[...],
             w2_ref[...], b2_ref[...], C, bs)
    hp = jax.nn.relu(jnp.dot(h, wl_ref[...], preferred_element_type=jnp.float32) + bl_ref[...])
    up = jnp.dot(hp, wu_ref[...], preferred_element_type=jnp.float32) + bu_ref[...]
    o_ref[...] = jax.nn.relu(f1_ref[...] + up)


def _perm_wl(wl, C):
    c_out = wl.shape[1]
    return wl.reshape(C, 4, c_out).transpose(1, 0, 2).reshape(4 * C, c_out)


def _full(shape):
    return pl.BlockSpec(shape, lambda g: tuple(0 for _ in shape))


def _conv_interp(rows, nx_flat, p, down, C, bs=256):
    BS = nx_flat.shape[0]
    D = rows.shape[2]
    (w1, b1), (w2, b2), (wl, bl) = p["w1"], p["w2"], p["wl"]
    wd, bd = down
    c_out, mid = wl.shape[1], wd.shape[1]
    wlp = _perm_wl(wl, C)
    body = functools.partial(_conv_interp_body, C=C, bs=bs)
    f1, hd = pl.pallas_call(
        body,
        grid=(BS // bs,),
        in_specs=[
            pl.BlockSpec((K, bs, D), lambda g: (0, g, 0)),
            pl.BlockSpec((bs, 3), lambda g: (g, 0)),
            _full(w1.shape), _full(b1.shape), _full(w2.shape), _full(b2.shape),
            _full(wlp.shape), _full(bl.shape), _full(wd.shape), _full(bd.shape),
        ],
        out_specs=[
            pl.BlockSpec((bs, c_out), lambda g: (g, 0)),
            pl.BlockSpec((bs, mid), lambda g: (g, 0)),
        ],
        out_shape=[
            jax.ShapeDtypeStruct((BS, c_out), jnp.float32),
            jax.ShapeDtypeStruct((BS, mid), jnp.float32),
        ],
    )(rows, nx_flat, w1, b1, w2, b2, wlp, bl, wd, bd)
    return f1, hd


def _conv_pc(rows, nx_flat, f1, p, up, C, bs=256):
    BS = nx_flat.shape[0]
    D = rows.shape[2]
    (w1, b1), (w2, b2), (wl, bl) = p["w1"], p["w2"], p["wl"]
    wu, bu = up
    c_out = wu.shape[1]
    wlp = _perm_wl(wl, C)
    body = functools.partial(_conv_pc_body, C=C, bs=bs)
    out = pl.pallas_call(
        body,
        grid=(BS // bs,),
        in_specs=[
            pl.BlockSpec((K, bs, D), lambda g: (0, g, 0)),
            pl.BlockSpec((bs, 3), lambda g: (g, 0)),
            pl.BlockSpec((bs, c_out), lambda g: (g, 0)),
            _full(w1.shape), _full(b1.shape), _full(w2.shape), _full(b2.shape),
            _full(wlp.shape), _full(bl.shape), _full(wu.shape), _full(bu.shape),
        ],
        out_specs=pl.BlockSpec((bs, c_out), lambda g: (g, 0)),
        out_shape=jax.ShapeDtypeStruct((BS, c_out), jnp.float32),
    )(rows, nx_flat, f1, w1, b1, w2, b2, wlp, bl, wu, bu)
    return out


def _head_body(xyz_ref, f_ref, w1x_ref, w1f_ref, b1_ref, w2_ref, b2_ref, o_ref):
    h = jax.nn.relu(
        jnp.dot(xyz_ref[0], w1x_ref[...], preferred_element_type=jnp.float32)
        + jnp.dot(f_ref[0], w1f_ref[...], preferred_element_type=jnp.float32)
        + b1_ref[...])
    h = jnp.dot(h, w2_ref[...], preferred_element_type=jnp.float32) + b2_ref[...]
    o_ref[0] = jnp.max(h, axis=0, keepdims=True)


def _head(xyz4, feats, gparams):
    B, S, Cf = feats.shape
    (W1, b1), (W2, b2) = gparams
    W1x, W1f = W1[:3], W1[3:]
    out = pl.pallas_call(
        _head_body,
        grid=(B,),
        in_specs=[
            pl.BlockSpec((1, S, 3), lambda b: (b, 0, 0)),
            pl.BlockSpec((1, S, Cf), lambda b: (b, 0, 0)),
            _full(W1x.shape), _full(W1f.shape), _full(b1.shape),
            _full(W2.shape), _full(b2.shape),
        ],
        out_specs=pl.BlockSpec((1, 1, W2.shape[1]), lambda b: (b, 0, 0)),
        out_shape=jax.ShapeDtypeStruct((B, 1, W2.shape[1]), jnp.float32),
    )(xyz4, feats, W1x, W1f, b1, W2, b2)
    return out


def kernel(xyz_0, xyz_1, xyz_2, xyz_3, xyz_4, normal_0, normal_1, normal_2, normal_3, normal_4, params):
    xyzs = [xyz_0, xyz_1, xyz_2, xyz_3, xyz_4]
    B = xyz_0.shape[0]
    feats = _stem(xyzs[0], params["pw"])

    for i, lp in enumerate(params["layers"]):
        xyz, new_xyz = xyzs[i], xyzs[i + 1]
        N, S = xyz.shape[1], new_xyz.shape[1]
        C = feats.shape[2]
        nx_flat = new_xyz.reshape(B * S, 3)

        idx = _knn(new_xyz, xyz, N)
        nn_idx = _knn(new_xyz, new_xyz, S)

        D1 = _round16(C + 3)
        tbl = _pad_cols(jnp.concatenate([feats, xyz], -1), D1).reshape(B * N, D1)
        rows = _sc_gather(tbl, idx.reshape(K * B * S)).reshape(K, B * S, D1)
        f1, hd = _conv_interp(rows, nx_flat, lp["interp"], lp["down"], C)

        mid = hd.shape[1]
        D2 = _round16(mid + 3)
        tbl2 = _pad_cols(
            jnp.concatenate([hd.reshape(B, S, mid), new_xyz], -1), D2
        ).reshape(B * S, D2)
        rows2 = _sc_gather(tbl2, nn_idx.reshape(K * B * S)).reshape(K, B * S, D2)
        out = _conv_pc(rows2, nx_flat, f1, lp["pc"], lp["up"], mid)
        feats = out.reshape(B, S, -1)

    local = feats
    g = _head(xyzs[-1], feats, params["global"])
    return local, g


# speedup vs baseline: 1.4488x; 1.4488x over previous
import functools

import jax
import jax.numpy as jnp
from jax import lax
from jax.experimental import pallas as pl
from jax.experimental.pallas import tpu as pltpu
from jax.experimental.pallas import tpu_sc as plsc

K = 16


def _stem_body(x_ref, w0_ref, b0_ref, w1_ref, b1_ref, w2_ref, b2_ref, o_ref):
    x = x_ref[...]
    h = jax.nn.relu(jnp.dot(x, w0_ref[...], preferred_element_type=jnp.float32) + b0_ref[...])
    h = jax.nn.relu(jnp.dot(h, w1_ref[...], preferred_element_type=jnp.float32) + b1_ref[...])
    h = jax.nn.relu(jnp.dot(h, w2_ref[...], preferred_element_type=jnp.float32) + b2_ref[...])
    o_ref[...] = h


def _stem(xyz0, pw):
    B, N, _ = xyz0.shape
    (w0, b0), (w1, b1), (w2, b2) = pw
    x = xyz0.reshape(B * N, 3)
    out = pl.pallas_call(
        _stem_body,
        out_shape=jax.ShapeDtypeStruct((B * N, 16), jnp.float32),
    )(x, w0, b0, w1, b1, w2, b2)
    return out.reshape(B, N, 16)


def _ce(v, ix, i, j):
    p = v[i] > v[j]
    vi = jnp.where(p, v[j], v[i])
    vj = jnp.where(p, v[i], v[j])
    xi = jnp.where(p, ix[j], ix[i])
    xj = jnp.where(p, ix[i], ix[j])
    v[i], v[j], ix[i], ix[j] = vi, vj, xi, xj


def _bitonic_sort16(v, ix):
    for k in (2, 4, 8, 16):
        step = k // 2
        while step >= 1:
            for i in range(16):
                l = i ^ step
                if l > i:
                    if (i & k) == 0:
                        _ce(v, ix, i, l)
                    else:
                        _ce(v, ix, l, i)
            step //= 2


def _bitonic_merge16(v, ix):
    for step in (8, 4, 2, 1):
        for i in range(16):
            l = i ^ step
            if l > i:
                _ce(v, ix, i, l)


def _knn_body(qT_ref, r_ref, o_ref, *, N, bs, n_off_stride):
    b = pl.program_id(0)
    qT = qT_ref[0]
    r = r_ref[0]
    rr = jnp.sum(r * r, axis=1, keepdims=True)
    qq = jnp.sum(qT * qT, axis=0, keepdims=True)
    dT = rr - 2.0 * jnp.dot(r, qT, preferred_element_type=jnp.float32) + qq

    G = N // 16
    base = b * n_off_stride
    giota = lax.broadcasted_iota(jnp.int32, (G, bs), 0)
    v = [dT[j * G:(j + 1) * G, :] for j in range(16)]
    ix = [giota + (j * G + base) for j in range(16)]

    _bitonic_sort16(v, ix)

    g = G
    while g > 1:
        h = g // 2
        a_v = [v[j][:h, :] for j in range(16)]
        a_x = [ix[j][:h, :] for j in range(16)]
        b_v = [v[j][h:, :] for j in range(16)]
        b_x = [ix[j][h:, :] for j in range(16)]
        nv, nx = [], []
        for j in range(16):
            bv, bx = b_v[15 - j], b_x[15 - j]
            p = a_v[j] > bv
            nv.append(jnp.where(p, bv, a_v[j]))
            nx.append(jnp.where(p, bx, a_x[j]))
        v, ix = nv, nx
        _bitonic_merge16(v, ix)
        g = h

    o_ref[...] = jnp.concatenate(ix, axis=0).astype(jnp.int32)


def _knn(q, r, flat_offset_stride, bs=256):
    B, S, _ = q.shape
    N = r.shape[1]
    qT = q.transpose(0, 2, 1)
    body = functools.partial(_knn_body, N=N, bs=bs, n_off_stride=flat_offset_stride)
    out = pl.pallas_call(
        body,
        grid=(B, S // bs),
        in_specs=[
            pl.BlockSpec((1, 3, bs), lambda b, s: (b, 0, s)),
            pl.BlockSpec((1, N, 3), lambda b, s: (b, 0, 0)),
        ],
        out_specs=pl.BlockSpec((K, bs), lambda b, s, _S=S, _bs=bs: (0, b * (_S // _bs) + s)),
        out_shape=jax.ShapeDtypeStruct((K, B * S), jnp.int32),
    )(qT, r)
    return out


def _sc_gather(table, idx):
    R, D = table.shape
    M = idx.shape[0]
    NW = 32
    assert M % NW == 0
    b_per_w = M // NW
    chunk = b_per_w
    while chunk * D * 4 > 256 * 1024:
        chunk //= 2
    assert b_per_w % chunk == 0 and chunk % 8 == 0
    n_ch = b_per_w // chunk
    mesh = plsc.VectorSubcoreMesh(core_axis_name="c", subcore_axis_name="s")

    def body(table_hbm, idx_hbm, out_hbm, idx_v, rows_v, sem):
        wid = lax.axis_index("s") * 2 + lax.axis_index("c")
        base = wid * b_per_w
        pltpu.sync_copy(idx_hbm.at[pl.ds(base, b_per_w)], idx_v)
        for ch in range(n_ch):
            pltpu.async_copy(
                table_hbm.at[idx_v.at[pl.ds(ch * chunk, chunk)]], rows_v, sem
            ).wait()
            pltpu.sync_copy(rows_v, out_hbm.at[pl.ds(base + ch * chunk, chunk)])

    f = pl.kernel(
        body,
        out_type=jax.ShapeDtypeStruct((M, D), jnp.float32),
        mesh=mesh,
        compiler_params=pltpu.CompilerParams(use_tc_tiling_on_sc=False),
        scratch_types=[
            pltpu.VMEM((b_per_w,), jnp.int32),
            pltpu.VMEM((chunk, D), jnp.float32),
            pltpu.SemaphoreType.DMA,
        ],
    )
    return f(table, idx)


def _pad_cols(x, D):
    pad = D - x.shape[-1]
    if pad:
        x = jnp.concatenate([x, jnp.zeros(x.shape[:-1] + (pad,), x.dtype)], -1)
    return x


def _round16(n):
    return (n + 15) // 16 * 16


def _agg_t(rows3, nxT, w1_ref, b1_ref, w2_ref, b2_ref, C, bs):
    rowsT = jnp.transpose(rows3, (0, 2, 1))
    gxT = rowsT[:, C:C + 3, :] - nxT[None, :, :]
    u = [jax.nn.relu(gxT[:, 0:1, :] * w1_ref[0, m]
                     + gxT[:, 1:2, :] * w1_ref[1, m]
                     + gxT[:, 2:3, :] * w1_ref[2, m]
                     + b1_ref[m]) for m in range(4)]
    v = [jax.nn.relu(u[0] * w2_ref[0, m] + u[1] * w2_ref[1, m]
                     + u[2] * w2_ref[2, m] + u[3] * w2_ref[3, m]
                     + b2_ref[m]) for m in range(4)]
    gfT = rowsT[:, :C, :]
    hs = [jnp.sum(gfT * vm, axis=0) for vm in v]
    return jnp.concatenate(hs, axis=0)


def _mm(a, b):
    return jnp.dot(a, b, preferred_element_type=jnp.float32)


def _conv_interp_body(rows_ref, nxT_ref, w1_ref, b1_ref, w2_ref, b2_ref,
                      wlT_ref, blT_ref, wdT_ref, bdT_ref, f1T_ref, hd_ref, *, C, bs):
    hT = _agg_t(rows_ref[...], nxT_ref[...], w1_ref, b1_ref, w2_ref, b2_ref, C, bs)
    f1T = jax.nn.relu(_mm(wlT_ref[...], hT) + blT_ref[...])
    f1T_ref[...] = f1T
    hd_ref[...] = jax.nn.relu(_mm(wdT_ref[...], f1T) + bdT_ref[...]).T


def _conv_pc_body(rows_ref, nxT_ref, f1T_ref, w1_ref, b1_ref, w2_ref, b2_ref,
                  wlT_ref, blT_ref, wuT_ref, buT_ref, o_ref, *, C, bs):
    hT = _agg_t(rows_ref[...], nxT_ref[...], w1_ref, b1_ref, w2_ref, b2_ref, C, bs)
    hpT = jax.nn.relu(_mm(wlT_ref[...], hT) + blT_ref[...])
    upT = _mm(wuT_ref[...], hpT) + buT_ref[...]
    o_ref[...] = jax.nn.relu(f1T_ref[...] + upT).T


def _perm_wl(wl, C):
    c_out = wl.shape[1]
    return wl.reshape(C, 4, c_out).transpose(1, 0, 2).reshape(4 * C, c_out)


def _full(shape):
    return pl.BlockSpec(shape, lambda g: tuple(0 for _ in shape))


def _smem():
    return pl.BlockSpec(memory_space=pltpu.SMEM)


def _conv_interp(rows, nxT_flat, p, down, C, bs=256):
    BS = nxT_flat.shape[1]
    D = rows.shape[2]
    (w1, b1), (w2, b2), (wl, bl) = p["w1"], p["w2"], p["wl"]
    wd, bd = down
    c_out, mid = wl.shape[1], wd.shape[1]
    wlT = _perm_wl(wl, C).T
    body = functools.partial(_conv_interp_body, C=C, bs=bs)
    f1T, hd = pl.pallas_call(
        body,
        grid=(BS // bs,),
        in_specs=[
            pl.BlockSpec((K, bs, D), lambda g: (0, g, 0)),
            pl.BlockSpec((3, bs), lambda g: (0, g)),
            _smem(), _smem(), _smem(), _smem(),
            _full(wlT.shape), _full((c_out, 1)), _full(wd.T.shape), _full((mid, 1)),
        ],
        out_specs=[
            pl.BlockSpec((c_out, bs), lambda g: (0, g)),
            pl.BlockSpec((bs, mid), lambda g: (g, 0)),
        ],
        out_shape=[
            jax.ShapeDtypeStruct((c_out, BS), jnp.float32),
            jax.ShapeDtypeStruct((BS, mid), jnp.float32),
        ],
    )(rows, nxT_flat, w1, b1, w2, b2, wlT, bl[:, None], wd.T, bd[:, None])
    return f1T, hd


def _conv_pc(rows, nxT_flat, f1T, p, up, C, bs=256):
    BS = nxT_flat.shape[1]
    D = rows.shape[2]
    (w1, b1), (w2, b2), (wl, bl) = p["w1"], p["w2"], p["wl"]
    wu, bu = up
    c_out = wu.shape[1]
    wlT = _perm_wl(wl, C).T
    body = functools.partial(_conv_pc_body, C=C, bs=bs)
    out = pl.pallas_call(
        body,
        grid=(BS // bs,),
        in_specs=[
            pl.BlockSpec((K, bs, D), lambda g: (0, g, 0)),
            pl.BlockSpec((3, bs), lambda g: (0, g)),
            pl.BlockSpec((c_out, bs), lambda g: (0, g)),
            _smem(), _smem(), _smem(), _smem(),
            _full(wlT.shape), _full((wl.shape[1], 1)), _full(wu.T.shape), _full((c_out, 1)),
        ],
        out_specs=pl.BlockSpec((bs, c_out), lambda g: (g, 0)),
        out_shape=jax.ShapeDtypeStruct((BS, c_out), jnp.float32),
    )(rows, nxT_flat, f1T, w1, b1, w2, b2, wlT, bl[:, None], wu.T, bu[:, None])
    return out


def _head_body(xyz_ref, f_ref, w1x_ref, w1f_ref, b1_ref, w2_ref, b2_ref, o_ref):
    h = jax.nn.relu(
        jnp.dot(xyz_ref[0], w1x_ref[...], preferred_element_type=jnp.float32)
        + jnp.dot(f_ref[0], w1f_ref[...], preferred_element_type=jnp.float32)
        + b1_ref[...])
    h = jnp.dot(h, w2_ref[...], preferred_element_type=jnp.float32) + b2_ref[...]
    o_ref[0] = jnp.max(h, axis=0, keepdims=True)


def _head(xyz4, feats, gparams):
    B, S, Cf = feats.shape
    (W1, b1), (W2, b2) = gparams
    W1x, W1f = W1[:3], W1[3:]
    out = pl.pallas_call(
        _head_body,
        grid=(B,),
        in_specs=[
            pl.BlockSpec((1, S, 3), lambda b: (b, 0, 0)),
            pl.BlockSpec((1, S, Cf), lambda b: (b, 0, 0)),
            _full(W1x.shape), _full(W1f.shape), _full(b1.shape),
            _full(W2.shape), _full(b2.shape),
        ],
        out_specs=pl.BlockSpec((1, 1, W2.shape[1]), lambda b: (b, 0, 0)),
        out_shape=jax.ShapeDtypeStruct((B, 1, W2.shape[1]), jnp.float32),
    )(xyz4, feats, W1x, W1f, b1, W2, b2)
    return out


def kernel(xyz_0, xyz_1, xyz_2, xyz_3, xyz_4, normal_0, normal_1, normal_2, normal_3, normal_4, params):
    xyzs = [xyz_0, xyz_1, xyz_2, xyz_3, xyz_4]
    B = xyz_0.shape[0]
    feats = _stem(xyzs[0], params["pw"])

    for i, lp in enumerate(params["layers"]):
        xyz, new_xyz = xyzs[i], xyzs[i + 1]
        N, S = xyz.shape[1], new_xyz.shape[1]
        C = feats.shape[2]
        nxT_flat = jnp.transpose(new_xyz, (2, 0, 1)).reshape(3, B * S)

        idx = _knn(new_xyz, xyz, N)
        nn_idx = _knn(new_xyz, new_xyz, S)

        D1 = _round16(C + 3)
        tbl = _pad_cols(jnp.concatenate([feats, xyz], -1), D1).reshape(B * N, D1)
        rows = _sc_gather(tbl, idx.reshape(K * B * S)).reshape(K, B * S, D1)
        f1T, hd = _conv_interp(rows, nxT_flat, lp["interp"], lp["down"], C)

        mid = hd.shape[1]
        D2 = _round16(mid + 3)
        tbl2 = _pad_cols(
            jnp.concatenate([hd.reshape(B, S, mid), new_xyz], -1), D2
        ).reshape(B * S, D2)
        rows2 = _sc_gather(tbl2, nn_idx.reshape(K * B * S)).reshape(K, B * S, D2)
        out = _conv_pc(rows2, nxT_flat, f1T, lp["pc"], lp["up"], mid)
        feats = out.reshape(B, S, -1)

    local = feats
    g = _head(xyzs[-1], feats, params["global"])
    return local, g


# speedup vs baseline: 1.4892x; 1.0279x over previous
import functools

import jax
import jax.numpy as jnp
from jax import lax
from jax.experimental import pallas as pl
from jax.experimental.pallas import tpu as pltpu
from jax.experimental.pallas import tpu_sc as plsc

K = 16


def _stem_body(x_ref, w0_ref, b0_ref, w1_ref, b1_ref, w2_ref, b2_ref, o_ref):
    x = x_ref[...]
    h = jax.nn.relu(jnp.dot(x, w0_ref[...], preferred_element_type=jnp.float32) + b0_ref[...])
    h = jax.nn.relu(jnp.dot(h, w1_ref[...], preferred_element_type=jnp.float32) + b1_ref[...])
    h = jax.nn.relu(jnp.dot(h, w2_ref[...], preferred_element_type=jnp.float32) + b2_ref[...])
    n = x.shape[0]
    o_ref[...] = jnp.concatenate(
        [h, x, jnp.zeros((n, 32 - 16 - 3), jnp.float32)], axis=1)


def _stem(xyz0, pw):
    B, N, _ = xyz0.shape
    (w0, b0), (w1, b1), (w2, b2) = pw
    x = xyz0.reshape(B * N, 3)
    return pl.pallas_call(
        _stem_body,
        out_shape=jax.ShapeDtypeStruct((B * N, 32), jnp.float32),
    )(x, w0, b0, w1, b1, w2, b2)


def _ce(v, ix, i, j):
    p = v[i] > v[j]
    vi = jnp.where(p, v[j], v[i])
    vj = jnp.where(p, v[i], v[j])
    xi = jnp.where(p, ix[j], ix[i])
    xj = jnp.where(p, ix[i], ix[j])
    v[i], v[j], ix[i], ix[j] = vi, vj, xi, xj


def _bitonic_sort16(v, ix):
    for k in (2, 4, 8, 16):
        step = k // 2
        while step >= 1:
            for i in range(16):
                l = i ^ step
                if l > i:
                    if (i & k) == 0:
                        _ce(v, ix, i, l)
                    else:
                        _ce(v, ix, l, i)
            step //= 2


def _bitonic_merge16(v, ix):
    for step in (8, 4, 2, 1):
        for i in range(16):
            l = i ^ step
            if l > i:
                _ce(v, ix, i, l)


def _knn_body(qT_ref, r_ref, o_ref, *, N, bs, n_off_stride):
    b = pl.program_id(0)
    qT = qT_ref[0]
    r = r_ref[0]
    rr = jnp.sum(r * r, axis=1, keepdims=True)
    qq = jnp.sum(qT * qT, axis=0, keepdims=True)
    dT = (qq - 2.0 * jnp.dot(r, qT, preferred_element_type=jnp.float32)) + rr

    G = N // 16
    base = b * n_off_stride
    giota = lax.broadcasted_iota(jnp.int32, (G, bs), 0)
    v = [dT[j * G:(j + 1) * G, :] for j in range(16)]
    ix = [giota + (j * G + base) for j in range(16)]

    _bitonic_sort16(v, ix)

    g = G
    while g > 1:
        h = g // 2
        a_v = [v[j][:h, :] for j in range(16)]
        a_x = [ix[j][:h, :] for j in range(16)]
        b_v = [v[j][h:, :] for j in range(16)]
        b_x = [ix[j][h:, :] for j in range(16)]
        nv, nx = [], []
        for j in range(16):
            bv, bx = b_v[15 - j], b_x[15 - j]
            p = a_v[j] > bv
            nv.append(jnp.where(p, bv, a_v[j]))
            nx.append(jnp.where(p, bx, a_x[j]))
        v, ix = nv, nx
        _bitonic_merge16(v, ix)
        g = h

    o_ref[...] = jnp.concatenate(ix, axis=0).astype(jnp.int32)


def _knn(q, r, flat_offset_stride, bs=256):
    B, S, _ = q.shape
    N = r.shape[1]
    qT = q.transpose(0, 2, 1)
    body = functools.partial(_knn_body, N=N, bs=bs, n_off_stride=flat_offset_stride)
    out = pl.pallas_call(
        body,
        grid=(B, S // bs),
        in_specs=[
            pl.BlockSpec((1, 3, bs), lambda b, s: (b, 0, s)),
            pl.BlockSpec((1, N, 3), lambda b, s: (b, 0, 0)),
        ],
        out_specs=pl.BlockSpec((K, bs), lambda b, s, _S=S, _bs=bs: (0, b * (_S // _bs) + s)),
        out_shape=jax.ShapeDtypeStruct((K, B * S), jnp.int32),
    )(qT, r)
    return out


def _sc_gather(table, idx):
    R, D = table.shape
    M = idx.shape[0]
    NW = 32
    assert M % NW == 0
    b_per_w = M // NW
    chunk = b_per_w
    while chunk * D * 4 > 256 * 1024:
        chunk //= 2
    assert b_per_w % chunk == 0 and chunk % 8 == 0
    n_ch = b_per_w // chunk
    mesh = plsc.VectorSubcoreMesh(core_axis_name="c", subcore_axis_name="s")

    def body(table_hbm, idx_hbm, out_hbm, idx_v, rows_v, sem):
        wid = lax.axis_index("s") * 2 + lax.axis_index("c")
        base = wid * b_per_w
        pltpu.sync_copy(idx_hbm.at[pl.ds(base, b_per_w)], idx_v)
        for ch in range(n_ch):
            pltpu.async_copy(
                table_hbm.at[idx_v.at[pl.ds(ch * chunk, chunk)]], rows_v, sem
            ).wait()
            pltpu.sync_copy(rows_v, out_hbm.at[pl.ds(base + ch * chunk, chunk)])

    f = pl.kernel(
        body,
        out_type=jax.ShapeDtypeStruct((M, D), jnp.float32),
        mesh=mesh,
        compiler_params=pltpu.CompilerParams(use_tc_tiling_on_sc=False),
        scratch_types=[
            pltpu.VMEM((b_per_w,), jnp.int32),
            pltpu.VMEM((chunk, D), jnp.float32),
            pltpu.SemaphoreType.DMA,
        ],
    )
    return f(table, idx)


def _pad_cols(x, D):
    pad = D - x.shape[-1]
    if pad:
        x = jnp.concatenate([x, jnp.zeros(x.shape[:-1] + (pad,), x.dtype)], -1)
    return x


def _round16(n):
    return (n + 15) // 16 * 16


def _agg_t(rows3, nxT, w1_ref, b1_ref, w2_ref, b2_ref, C, bs):
    rowsT = jnp.transpose(rows3, (0, 2, 1))
    gxT = rowsT[:, C:C + 3, :] - nxT[None, :, :]
    u = [jax.nn.relu(gxT[:, 0:1, :] * w1_ref[0, m]
                     + gxT[:, 1:2, :] * w1_ref[1, m]
                     + gxT[:, 2:3, :] * w1_ref[2, m]
                     + b1_ref[m]) for m in range(4)]
    v = [jax.nn.relu(u[0] * w2_ref[0, m] + u[1] * w2_ref[1, m]
                     + u[2] * w2_ref[2, m] + u[3] * w2_ref[3, m]
                     + b2_ref[m]) for m in range(4)]
    gfT = rowsT[:, :C, :]
    hs = [jnp.sum(gfT * vm, axis=0) for vm in v]
    return jnp.concatenate(hs, axis=0)


def _mm(a, b):
    return jnp.dot(a, b, preferred_element_type=jnp.float32)


def _conv_interp_body(rows_ref, nxT_ref, w1_ref, b1_ref, w2_ref, b2_ref,
                      wlT_ref, blT_ref, wdT_ref, bdT_ref, f1T_ref, tbl_ref,
                      *, C, bs, D2):
    nxT = nxT_ref[...]
    hT = _agg_t(rows_ref[...], nxT, w1_ref, b1_ref, w2_ref, b2_ref, C, bs)
    f1T = jax.nn.relu(_mm(wlT_ref[...], hT) + blT_ref[...])
    f1T_ref[...] = f1T
    hdT = jax.nn.relu(_mm(wdT_ref[...], f1T) + bdT_ref[...])
    mid = hdT.shape[0]
    tbl_ref[...] = jnp.concatenate(
        [hdT, nxT, jnp.zeros((D2 - mid - 3, bs), jnp.float32)], axis=0).T


def _conv_pc_body(rows_ref, nxT_ref, f1T_ref, w1_ref, b1_ref, w2_ref, b2_ref,
                  wlT_ref, blT_ref, wuT_ref, buT_ref, o_ref, *, C, bs, Dn):
    nxT = nxT_ref[...]
    hT = _agg_t(rows_ref[...], nxT, w1_ref, b1_ref, w2_ref, b2_ref, C, bs)
    hpT = jax.nn.relu(_mm(wlT_ref[...], hT) + blT_ref[...])
    upT = _mm(wuT_ref[...], hpT) + buT_ref[...]
    outT = jax.nn.relu(f1T_ref[...] + upT)
    if Dn is None:
        o_ref[...] = outT.T
    else:
        c_out = outT.shape[0]
        o_ref[...] = jnp.concatenate(
            [outT, nxT, jnp.zeros((Dn - c_out - 3, bs), jnp.float32)], axis=0).T


def _perm_wl(wl, C):
    c_out = wl.shape[1]
    return wl.reshape(C, 4, c_out).transpose(1, 0, 2).reshape(4 * C, c_out)


def _full(shape):
    return pl.BlockSpec(shape, lambda g: tuple(0 for _ in shape))


def _smem():
    return pl.BlockSpec(memory_space=pltpu.SMEM)


def _conv_interp(rows, nxT_flat, p, down, C, D2, bs=256):
    BS = nxT_flat.shape[1]
    D = rows.shape[2]
    (w1, b1), (w2, b2), (wl, bl) = p["w1"], p["w2"], p["wl"]
    wd, bd = down
    c_out, mid = wl.shape[1], wd.shape[1]
    wlT = _perm_wl(wl, C).T
    body = functools.partial(_conv_interp_body, C=C, bs=bs, D2=D2)
    f1T, tbl2 = pl.pallas_call(
        body,
        grid=(BS // bs,),
        in_specs=[
            pl.BlockSpec((K, bs, D), lambda g: (0, g, 0)),
            pl.BlockSpec((3, bs), lambda g: (0, g)),
            _smem(), _smem(), _smem(), _smem(),
            _full(wlT.shape), _full((c_out, 1)), _full(wd.T.shape), _full((mid, 1)),
        ],
        out_specs=[
            pl.BlockSpec((c_out, bs), lambda g: (0, g)),
            pl.BlockSpec((bs, D2), lambda g: (g, 0)),
        ],
        out_shape=[
            jax.ShapeDtypeStruct((c_out, BS), jnp.float32),
            jax.ShapeDtypeStruct((BS, D2), jnp.float32),
        ],
    )(rows, nxT_flat, w1, b1, w2, b2, wlT, bl[:, None], wd.T, bd[:, None])
    return f1T, tbl2


def _conv_pc(rows, nxT_flat, f1T, p, up, C, Dn, bs=256):
    BS = nxT_flat.shape[1]
    D = rows.shape[2]
    (w1, b1), (w2, b2), (wl, bl) = p["w1"], p["w2"], p["wl"]
    wu, bu = up
    c_out = wu.shape[1]
    Dout = c_out if Dn is None else Dn
    wlT = _perm_wl(wl, C).T
    body = functools.partial(_conv_pc_body, C=C, bs=bs, Dn=Dn)
    out = pl.pallas_call(
        body,
        grid=(BS // bs,),
        in_specs=[
            pl.BlockSpec((K, bs, D), lambda g: (0, g, 0)),
            pl.BlockSpec((3, bs), lambda g: (0, g)),
            pl.BlockSpec((c_out, bs), lambda g: (0, g)),
            _smem(), _smem(), _smem(), _smem(),
            _full(wlT.shape), _full((wl.shape[1], 1)), _full(wu.T.shape), _full((c_out, 1)),
        ],
        out_specs=pl.BlockSpec((bs, Dout), lambda g: (g, 0)),
        out_shape=jax.ShapeDtypeStruct((BS, Dout), jnp.float32),
    )(rows, nxT_flat, f1T, w1, b1, w2, b2, wlT, bl[:, None], wu.T, bu[:, None])
    return out


def _head_body(xyz_ref, f_ref, w1x_ref, w1f_ref, b1_ref, w2_ref, b2_ref, o_ref):
    h = jax.nn.relu(
        jnp.dot(xyz_ref[0], w1x_ref[...], preferred_element_type=jnp.float32)
        + jnp.dot(f_ref[0], w1f_ref[...], preferred_element_type=jnp.float32)
        + b1_ref[...])
    h = jnp.dot(h, w2_ref[...], preferred_element_type=jnp.float32) + b2_ref[...]
    o_ref[0] = jnp.max(h, axis=0, keepdims=True)


def _head(xyz4, feats, gparams):
    B, S, Cf = feats.shape
    (W1, b1), (W2, b2) = gparams
    W1x, W1f = W1[:3], W1[3:]
    out = pl.pallas_call(
        _head_body,
        grid=(B,),
        in_specs=[
            pl.BlockSpec((1, S, 3), lambda b: (b, 0, 0)),
            pl.BlockSpec((1, S, Cf), lambda b: (b, 0, 0)),
            _full(W1x.shape), _full(W1f.shape), _full(b1.shape),
            _full(W2.shape), _full(b2.shape),
        ],
        out_specs=pl.BlockSpec((1, 1, W2.shape[1]), lambda b: (b, 0, 0)),
        out_shape=jax.ShapeDtypeStruct((B, 1, W2.shape[1]), jnp.float32),
    )(xyz4, feats, W1x, W1f, b1, W2, b2)
    return out


def kernel(xyz_0, xyz_1, xyz_2, xyz_3, xyz_4, normal_0, normal_1, normal_2, normal_3, normal_4, params):
    xyzs = [xyz_0, xyz_1, xyz_2, xyz_3, xyz_4]
    B = xyz_0.shape[0]
    Cs = [16, 32, 64, 128]
    mids = [8, 16, 32, 64]
    D1s = [_round16(c + 3) for c in Cs]
    D2s = [_round16(m + 3) for m in mids]

    idxs, nn_idxs, nxTs = [], [], []
    for i in range(4):
        xyz, new_xyz = xyzs[i], xyzs[i + 1]
        N, S = xyz.shape[1], new_xyz.shape[1]
        idxs.append(_knn(new_xyz, xyz, N).reshape(K * B * S))
        nn_idxs.append(_knn(new_xyz, new_xyz, S).reshape(K * B * S))
        nxTs.append(jnp.transpose(new_xyz, (2, 0, 1)).reshape(3, B * S))

    tbl = _stem(xyzs[0], params["pw"])
    for i, lp in enumerate(params["layers"]):
        S = xyzs[i + 1].shape[1]
        C, mid, D1, D2 = Cs[i], mids[i], D1s[i], D2s[i]
        Dn = D1s[i + 1] if i + 1 < 4 else None
        rows = _sc_gather(tbl, idxs[i]).reshape(K, B * S, D1)
        f1T, tbl2 = _conv_interp(rows, nxTs[i], lp["interp"], lp["down"], C, D2)
        rows2 = _sc_gather(tbl2, nn_idxs[i]).reshape(K, B * S, D2)
        tbl = _conv_pc(rows2, nxTs[i], f1T, lp["pc"], lp["up"], mid, Dn)

    local = tbl.reshape(B, 256, 256)
    g = _head(xyzs[-1], local, params["global"])
    return local, g


# speedup vs baseline: 1.5797x; 1.0608x over previous
import functools

import jax
import jax.numpy as jnp
from jax import lax
from jax.experimental import pallas as pl
from jax.experimental.pallas import tpu as pltpu
from jax.experimental.pallas import tpu_sc as plsc

K = 16


def _stem_body(x_ref, w0_ref, b0_ref, w1_ref, b1_ref, w2_ref, b2_ref, o_ref):
    x = x_ref[...]
    h = jax.nn.relu(jnp.dot(x, w0_ref[...], preferred_element_type=jnp.float32) + b0_ref[...])
    h = jax.nn.relu(jnp.dot(h, w1_ref[...], preferred_element_type=jnp.float32) + b1_ref[...])
    h = jax.nn.relu(jnp.dot(h, w2_ref[...], preferred_element_type=jnp.float32) + b2_ref[...])
    n = x.shape[0]
    o_ref[...] = jnp.concatenate(
        [h, x, jnp.zeros((n, 32 - 16 - 3), jnp.float32)], axis=1)


def _stem(xyz0, pw):
    B, N, _ = xyz0.shape
    (w0, b0), (w1, b1), (w2, b2) = pw
    x = xyz0.reshape(B * N, 3)
    return pl.pallas_call(
        _stem_body,
        out_shape=jax.ShapeDtypeStruct((B * N, 32), jnp.float32),
    )(x, w0, b0, w1, b1, w2, b2)


def _ce(v, ix, i, j):
    p = v[i] > v[j]
    vi = jnp.where(p, v[j], v[i])
    vj = jnp.where(p, v[i], v[j])
    xi = jnp.where(p, ix[j], ix[i])
    xj = jnp.where(p, ix[i], ix[j])
    v[i], v[j], ix[i], ix[j] = vi, vj, xi, xj


def _batcher_pairs16():
    n, pairs, p = 16, [], 1
    while p < n:
        k = p
        while k >= 1:
            for j in range(k % p, n - k, 2 * k):
                for i in range(0, min(k, n - j - k)):
                    if (i + j) // (2 * p) == (i + j + k) // (2 * p):
                        pairs.append((i + j, i + j + k))
            k //= 2
        p *= 2
    return pairs


_SORT16_PAIRS = _batcher_pairs16()


def _bitonic_sort16(v, ix):
    for i, l in _SORT16_PAIRS:
        _ce(v, ix, i, l)


def _bitonic_merge16(v, ix):
    for step in (8, 4, 2, 1):
        for i in range(16):
            l = i ^ step
            if l > i:
                _ce(v, ix, i, l)


def _knn_body(qT_ref, r_ref, o_ref, *, N, bs, n_off_stride):
    b = pl.program_id(0)
    qT = qT_ref[0]
    r = r_ref[0]
    rr = jnp.sum(r * r, axis=1, keepdims=True)
    qq = jnp.sum(qT * qT, axis=0, keepdims=True)
    dT = (qq - 2.0 * jnp.dot(r, qT, preferred_element_type=jnp.float32)) + rr

    G = N // 16
    base = b * n_off_stride
    giota = lax.broadcasted_iota(jnp.int32, (G, bs), 0)
    v = [dT[j * G:(j + 1) * G, :] for j in range(16)]
    ix = [giota + (j * G + base) for j in range(16)]

    _bitonic_sort16(v, ix)

    g = G
    while g > 1:
        h = g // 2
        a_v = [v[j][:h, :] for j in range(16)]
        a_x = [ix[j][:h, :] for j in range(16)]
        b_v = [v[j][h:, :] for j in range(16)]
        b_x = [ix[j][h:, :] for j in range(16)]
        nv, nx = [], []
        for j in range(16):
            bv, bx = b_v[15 - j], b_x[15 - j]
            p = a_v[j] > bv
            nv.append(jnp.where(p, bv, a_v[j]))
            nx.append(jnp.where(p, bx, a_x[j]))
        v, ix = nv, nx
        _bitonic_merge16(v, ix)
        g = h

    o_ref[...] = jnp.concatenate(ix, axis=0).astype(jnp.int32)


def _knn(q, r, flat_offset_stride, bs=256):
    B, S, _ = q.shape
    N = r.shape[1]
    qT = q.transpose(0, 2, 1)
    body = functools.partial(_knn_body, N=N, bs=bs, n_off_stride=flat_offset_stride)
    out = pl.pallas_call(
        body,
        grid=(B, S // bs),
        in_specs=[
            pl.BlockSpec((1, 3, bs), lambda b, s: (b, 0, s)),
            pl.BlockSpec((1, N, 3), lambda b, s: (b, 0, 0)),
        ],
        out_specs=pl.BlockSpec((K, bs), lambda b, s, _S=S, _bs=bs: (0, b * (_S // _bs) + s)),
        out_shape=jax.ShapeDtypeStruct((K, B * S), jnp.int32),
    )(qT, r)
    return out


def _sc_gather(table, idx):
    R, D = table.shape
    M = idx.shape[0]
    NW = 32
    assert M % NW == 0
    b_per_w = M // NW
    chunk = b_per_w
    while chunk * D * 4 > 256 * 1024:
        chunk //= 2
    assert b_per_w % chunk == 0 and chunk % 8 == 0
    n_ch = b_per_w // chunk
    mesh = plsc.VectorSubcoreMesh(core_axis_name="c", subcore_axis_name="s")

    def body(table_hbm, idx_hbm, out_hbm, idx_v, rows_v, sem):
        wid = lax.axis_index("s") * 2 + lax.axis_index("c")
        base = wid * b_per_w
        pltpu.sync_copy(idx_hbm.at[pl.ds(base, b_per_w)], idx_v)
        for ch in range(n_ch):
            pltpu.async_copy(
                table_hbm.at[idx_v.at[pl.ds(ch * chunk, chunk)]], rows_v, sem
            ).wait()
            pltpu.sync_copy(rows_v, out_hbm.at[pl.ds(base + ch * chunk, chunk)])

    f = pl.kernel(
        body,
        out_type=jax.ShapeDtypeStruct((M, D), jnp.float32),
        mesh=mesh,
        compiler_params=pltpu.CompilerParams(use_tc_tiling_on_sc=False),
        scratch_types=[
            pltpu.VMEM((b_per_w,), jnp.int32),
            pltpu.VMEM((chunk, D), jnp.float32),
            pltpu.SemaphoreType.DMA,
        ],
    )
    return f(table, idx)


def _pad_cols(x, D):
    pad = D - x.shape[-1]
    if pad:
        x = jnp.concatenate([x, jnp.zeros(x.shape[:-1] + (pad,), x.dtype)], -1)
    return x


def _round16(n):
    return (n + 15) // 16 * 16


def _agg_t(rows3, nxT, w1_ref, b1_ref, w2_ref, b2_ref, C, bs):
    rowsT = jnp.transpose(rows3, (0, 2, 1))
    gxT = rowsT[:, C:C + 3, :] - nxT[None, :, :]
    u = [jax.nn.relu(gxT[:, 0:1, :] * w1_ref[0, m]
                     + gxT[:, 1:2, :] * w1_ref[1, m]
                     + gxT[:, 2:3, :] * w1_ref[2, m]
                     + b1_ref[m]) for m in range(4)]
    v = [jax.nn.relu(u[0] * w2_ref[0, m] + u[1] * w2_ref[1, m]
                     + u[2] * w2_ref[2, m] + u[3] * w2_ref[3, m]
                     + b2_ref[m]) for m in range(4)]
    gfT = rowsT[:, :C, :]
    hs = [jnp.sum(gfT * vm, axis=0) for vm in v]
    return jnp.concatenate(hs, axis=0)


def _mm(a, b):
    return jnp.dot(a, b, preferred_element_type=jnp.float32)


def _conv_interp_body(rows_ref, nxT_ref, w1_ref, b1_ref, w2_ref, b2_ref,
                      wlT_ref, blT_ref, wdT_ref, bdT_ref, f1T_ref, tbl_ref,
                      *, C, bs, D2):
    nxT = nxT_ref[...]
    hT = _agg_t(rows_ref[...], nxT, w1_ref, b1_ref, w2_ref, b2_ref, C, bs)
    f1T = jax.nn.relu(_mm(wlT_ref[...], hT) + blT_ref[...])
    f1T_ref[...] = f1T
    hdT = jax.nn.relu(_mm(wdT_ref[...], f1T) + bdT_ref[...])
    mid = hdT.shape[0]
    tbl_ref[...] = jnp.concatenate(
        [hdT, nxT, jnp.zeros((D2 - mid - 3, bs), jnp.float32)], axis=0).T


def _conv_pc_body(rows_ref, nxT_ref, f1T_ref, w1_ref, b1_ref, w2_ref, b2_ref,
                  wlT_ref, blT_ref, wuT_ref, buT_ref, o_ref, *, C, bs, Dn):
    nxT = nxT_ref[...]
    hT = _agg_t(rows_ref[...], nxT, w1_ref, b1_ref, w2_ref, b2_ref, C, bs)
    hpT = jax.nn.relu(_mm(wlT_ref[...], hT) + blT_ref[...])
    upT = _mm(wuT_ref[...], hpT) + buT_ref[...]
    outT = jax.nn.relu(f1T_ref[...] + upT)
    if Dn is None:
        o_ref[...] = outT.T
    else:
        c_out = outT.shape[0]
        o_ref[...] = jnp.concatenate(
            [outT, nxT, jnp.zeros((Dn - c_out - 3, bs), jnp.float32)], axis=0).T


def _perm_wl(wl, C):
    c_out = wl.shape[1]
    return wl.reshape(C, 4, c_out).transpose(1, 0, 2).reshape(4 * C, c_out)


def _full(shape):
    return pl.BlockSpec(shape, lambda g: tuple(0 for _ in shape))


def _smem():
    return pl.BlockSpec(memory_space=pltpu.SMEM)


def _conv_interp(rows, nxT_flat, p, down, C, D2, bs=256):
    BS = nxT_flat.shape[1]
    D = rows.shape[2]
    (w1, b1), (w2, b2), (wl, bl) = p["w1"], p["w2"], p["wl"]
    wd, bd = down
    c_out, mid = wl.shape[1], wd.shape[1]
    wlT = _perm_wl(wl, C).T
    body = functools.partial(_conv_interp_body, C=C, bs=bs, D2=D2)
    f1T, tbl2 = pl.pallas_call(
        body,
        grid=(BS // bs,),
        in_specs=[
            pl.BlockSpec((K, bs, D), lambda g: (0, g, 0)),
            pl.BlockSpec((3, bs), lambda g: (0, g)),
            _smem(), _smem(), _smem(), _smem(),
            _full(wlT.shape), _full((c_out, 1)), _full(wd.T.shape), _full((mid, 1)),
        ],
        out_specs=[
            pl.BlockSpec((c_out, bs), lambda g: (0, g)),
            pl.BlockSpec((bs, D2), lambda g: (g, 0)),
        ],
        out_shape=[
            jax.ShapeDtypeStruct((c_out, BS), jnp.float32),
            jax.ShapeDtypeStruct((BS, D2), jnp.float32),
        ],
    )(rows, nxT_flat, w1, b1, w2, b2, wlT, bl[:, None], wd.T, bd[:, None])
    return f1T, tbl2


def _conv_pc(rows, nxT_flat, f1T, p, up, C, Dn, bs=256):
    BS = nxT_flat.shape[1]
    D = rows.shape[2]
    (w1, b1), (w2, b2), (wl, bl) = p["w1"], p["w2"], p["wl"]
    wu, bu = up
    c_out = wu.shape[1]
    Dout = c_out if Dn is None else Dn
    wlT = _perm_wl(wl, C).T
    body = functools.partial(_conv_pc_body, C=C, bs=bs, Dn=Dn)
    out = pl.pallas_call(
        body,
        grid=(BS // bs,),
        in_specs=[
            pl.BlockSpec((K, bs, D), lambda g: (0, g, 0)),
            pl.BlockSpec((3, bs), lambda g: (0, g)),
            pl.BlockSpec((c_out, bs), lambda g: (0, g)),
            _smem(), _smem(), _smem(), _smem(),
            _full(wlT.shape), _full((wl.shape[1], 1)), _full(wu.T.shape), _full((c_out, 1)),
        ],
        out_specs=pl.BlockSpec((bs, Dout), lambda g: (g, 0)),
        out_shape=jax.ShapeDtypeStruct((BS, Dout), jnp.float32),
    )(rows, nxT_flat, f1T, w1, b1, w2, b2, wlT, bl[:, None], wu.T, bu[:, None])
    return out


def _head_body(xyz_ref, f_ref, w1x_ref, w1f_ref, b1_ref, w2_ref, b2_ref, o_ref):
    h = jax.nn.relu(
        jnp.dot(xyz_ref[0], w1x_ref[...], preferred_element_type=jnp.float32)
        + jnp.dot(f_ref[0], w1f_ref[...], preferred_element_type=jnp.float32)
        + b1_ref[...])
    h = jnp.dot(h, w2_ref[...], preferred_element_type=jnp.float32) + b2_ref[...]
    o_ref[0] = jnp.max(h, axis=0, keepdims=True)


def _head(xyz4, feats, gparams):
    B, S, Cf = feats.shape
    (W1, b1), (W2, b2) = gparams
    W1x, W1f = W1[:3], W1[3:]
    out = pl.pallas_call(
        _head_body,
        grid=(B,),
        in_specs=[
            pl.BlockSpec((1, S, 3), lambda b: (b, 0, 0)),
            pl.BlockSpec((1, S, Cf), lambda b: (b, 0, 0)),
            _full(W1x.shape), _full(W1f.shape), _full(b1.shape),
            _full(W2.shape), _full(b2.shape),
        ],
        out_specs=pl.BlockSpec((1, 1, W2.shape[1]), lambda b: (b, 0, 0)),
        out_shape=jax.ShapeDtypeStruct((B, 1, W2.shape[1]), jnp.float32),
    )(xyz4, feats, W1x, W1f, b1, W2, b2)
    return out


def kernel(xyz_0, xyz_1, xyz_2, xyz_3, xyz_4, normal_0, normal_1, normal_2, normal_3, normal_4, params):
    xyzs = [xyz_0, xyz_1, xyz_2, xyz_3, xyz_4]
    B = xyz_0.shape[0]
    Cs = [16, 32, 64, 128]
    mids = [8, 16, 32, 64]
    D1s = [_round16(c + 3) for c in Cs]
    D2s = [_round16(m + 3) for m in mids]

    idxs, nn_idxs, nxTs = [], [], []
    for i in range(4):
        xyz, new_xyz = xyzs[i], xyzs[i + 1]
        N, S = xyz.shape[1], new_xyz.shape[1]
        idxs.append(_knn(new_xyz, xyz, N).reshape(K * B * S))
        nn_idxs.append(_knn(new_xyz, new_xyz, S).reshape(K * B * S))
        nxTs.append(jnp.transpose(new_xyz, (2, 0, 1)).reshape(3, B * S))

    tbl = _stem(xyzs[0], params["pw"])
    for i, lp in enumerate(params["layers"]):
        S = xyzs[i + 1].shape[1]
        C, mid, D1, D2 = Cs[i], mids[i], D1s[i], D2s[i]
        Dn = D1s[i + 1] if i + 1 < 4 else None
        rows = _sc_gather(tbl, idxs[i]).reshape(K, B * S, D1)
        f1T, tbl2 = _conv_interp(rows, nxTs[i], lp["interp"], lp["down"], C, D2)
        rows2 = _sc_gather(tbl2, nn_idxs[i]).reshape(K, B * S, D2)
        tbl = _conv_pc(rows2, nxTs[i], f1T, lp["pc"], lp["up"], mid, Dn)

    local = tbl.reshape(B, 256, 256)
    g = _head(xyzs[-1], local, params["global"])
    return local, g
